# baseline (speedup 1.0000x reference)
import functools

import jax
import jax.numpy as jnp
from jax import lax
from jax.experimental import pallas as pl
from jax.experimental.pallas import tpu as pltpu
from jax.experimental.pallas import tpu_sc as plsc

N = 10000
E = 320000
D = 128
H = 128
R = 4
C = 16

NC = 2
NS = 16
B = 96
NBUF = 3
NB = 54
EPT = NB * B
EPR = E // R
EPR_PAD = NS * EPT
N_PAD = 10112
ROWS_PER_TILE = N_PAD // NS
ZCHUNKS = ROWS_PER_TILE // B


def _sc_aggregate_body(h_hbm, src_hbm, dst_hbm, zrow_hbm, out_hbm,
                       srcbuf, dstbuf, *bufs_and_sems):
    rows = bufs_and_sems[:NBUF]
    acc = bufs_and_sems[NBUF]
    gsems = bufs_and_sems[NBUF + 1:2 * NBUF + 1]
    ssems = bufs_and_sems[2 * NBUF + 1:]
    c = lax.axis_index("c")
    s = lax.axis_index("s")
    for j in range(R // NC):
        r = c * (R // NC) + j
        pltpu.sync_copy(zrow_hbm, rows[0])
        zbase = s * ROWS_PER_TILE
        for kz in range(ZCHUNKS):
            pltpu.sync_copy(rows[0], acc.at[pl.ds(zbase + kz * B, B)])
        rem = ROWS_PER_TILE - ZCHUNKS * B
        if rem:
            pltpu.sync_copy(rows[0].at[pl.ds(0, rem)],
                            acc.at[pl.ds(zbase + ZCHUNKS * B, rem)])
        plsc.subcore_barrier()
        ebase = (r * NS + s) * EPT
        pltpu.sync_copy(src_hbm.at[pl.ds(ebase, EPT)], srcbuf)
        pltpu.sync_copy(dst_hbm.at[r, s], dstbuf)
        for q in range(NBUF):
            pltpu.async_copy(h_hbm.at[srcbuf.at[pl.ds(q * B, B)]],
                             rows[q], gsems[q])

        def quad(i, carry):
            for q in range(NBUF):
                k = NBUF * i + q
                pltpu.make_async_copy(h_hbm.at[srcbuf.at[pl.ds(k * B, B)]],
                                      rows[q], gsems[q]).wait()
                pltpu.sync_copy(rows[q], acc.at[dstbuf.at[k]], add=True)

                @pl.when(k + NBUF < NB)
                def _():
                    pltpu.async_copy(
                        h_hbm.at[srcbuf.at[pl.ds((k + NBUF) * B, B)]],
                        rows[q], gsems[q])

            return carry

        lax.fori_loop(0, NB // NBUF, quad, 0)
        plsc.subcore_barrier()
        pltpu.sync_copy(acc.at[pl.ds(s * ROWS_PER_TILE, ROWS_PER_TILE)],
                        out_hbm.at[r, pl.ds(s * ROWS_PER_TILE, ROWS_PER_TILE)])
        plsc.subcore_barrier()


_sc_aggregate = functools.partial(
    pl.kernel,
    out_type=jax.ShapeDtypeStruct((R, N_PAD, D), jnp.float32),
    mesh=plsc.VectorSubcoreMesh(core_axis_name="c", subcore_axis_name="s"),
    scratch_types=(
        [
            pltpu.VMEM((EPT,), jnp.int32),
            pltpu.VMEM((NB, B), jnp.int32),
        ]
        + [pltpu.VMEM((B, D), jnp.float32) for _ in range(NBUF)]
        + [pltpu.VMEM_SHARED((N_PAD, D), jnp.float32)]
        + [pltpu.SemaphoreType.DMA for _ in range(2 * NBUF)]
    ),
)(_sc_aggregate_body)


BN = 1000


def _tc_layer_body(a_ref, h_ref, w_ref, l_ref, b_ref, o_ref):
    acc = jnp.dot(h_ref[...], l_ref[...], preferred_element_type=jnp.float32)
    for r in range(R):
        acc = acc + jnp.dot(a_ref[r], w_ref[r],
                            preferred_element_type=jnp.float32)
    o_ref[...] = jnp.maximum(acc + b_ref[...], 0.0)


def _tc_layer(A, h, W, loop, b):
    return pl.pallas_call(
        _tc_layer_body,
        grid=(N // BN,),
        in_specs=[
            pl.BlockSpec((R, BN, D), lambda i: (0, i, 0)),
            pl.BlockSpec((BN, D), lambda i: (i, 0)),
            pl.BlockSpec((R, D, H), lambda i: (0, 0, 0)),
            pl.BlockSpec((D, H), lambda i: (0, 0)),
            pl.BlockSpec((1, H), lambda i: (0, 0)),
        ],
        out_specs=pl.BlockSpec((BN, H), lambda i: (i, 0)),
        out_shape=jax.ShapeDtypeStruct((N, H), jnp.float32),
    )(A, h, W, loop, b)


def _tc_layer2_body(a_ref, h_ref, w_ref, l_ref, b_ref, wc_ref, bc_ref,
                    o_ref, m_ref):
    i = pl.program_id(0)
    acc = jnp.dot(h_ref[...], l_ref[...], preferred_element_type=jnp.float32)
    for r in range(R):
        acc = acc + jnp.dot(a_ref[r], w_ref[r],
                            preferred_element_type=jnp.float32)
    x = jnp.maximum(acc + b_ref[...], 0.0)
    bm = jnp.max(x, axis=0, keepdims=True)

    @pl.when(i == 0)
    def _():
        m_ref[...] = bm

    @pl.when(i > 0)
    def _():
        m_ref[...] = jnp.maximum(m_ref[...], bm)

    @pl.when(i == N // BN - 1)
    def _():
        o_ref[...] = jnp.dot(m_ref[...], wc_ref[...],
                             preferred_element_type=jnp.float32) + bc_ref[...]


def _tc_layer2(A, h, W, loop, b, Wc, bc):
    return pl.pallas_call(
        _tc_layer2_body,
        grid=(N // BN,),
        in_specs=[
            pl.BlockSpec((R, BN, D), lambda i: (0, i, 0)),
            pl.BlockSpec((BN, D), lambda i: (i, 0)),
            pl.BlockSpec((R, D, H), lambda i: (0, 0, 0)),
            pl.BlockSpec((D, H), lambda i: (0, 0)),
            pl.BlockSpec((1, H), lambda i: (0, 0)),
            pl.BlockSpec((H, C), lambda i: (0, 0)),
            pl.BlockSpec((1, C), lambda i: (0, 0)),
        ],
        out_specs=pl.BlockSpec((1, C), lambda i: (0, 0)),
        out_shape=jax.ShapeDtypeStruct((1, C), jnp.float32),
        scratch_shapes=[pltpu.VMEM((1, H), jnp.float32)],
    )(A, h, W, loop, b, Wc, bc)


def kernel(h, edge_index, W1, loop1, b1, W2, loop2, b2, Wc, bc):
    ei = edge_index.astype(jnp.int32)
    src4 = ei[0].reshape(EPR, R).T
    dst4 = ei[1].reshape(EPR, R).T
    src_p = jnp.pad(src4, ((0, 0), (0, EPR_PAD - EPR))).reshape(-1)
    dst_p = jnp.pad(dst4, ((0, 0), (0, EPR_PAD - EPR)),
                    constant_values=N).reshape(R, NS, NB, B)
    zrow = jnp.zeros((B, D), jnp.float32)
    b1r = b1.reshape(1, H)
    b2r = b2.reshape(1, H)
    bcr = bc.reshape(1, C)

    A1 = _sc_aggregate(h, src_p, dst_p, zrow)[:, :N, :]
    h1 = _tc_layer(A1, h, W1, loop1, b1r)
    A2 = _sc_aggregate(h1, src_p, dst_p, zrow)[:, :N, :]
    return _tc_layer2(A2, h1, W2, loop2, b2r, Wc, bcr)


# speedup vs baseline: 1.9917x; 1.9917x over previous
import functools

import jax
import jax.numpy as jnp
from jax import lax
from jax.experimental import pallas as pl
from jax.experimental.pallas import tpu as pltpu
from jax.experimental.pallas import tpu_sc as plsc

N = 10000
E = 320000
D = 128
H = 128
R = 4
C = 16

NC = 2
NS = 16
B = 80
NBUF = 3
NB = 63
EPT = NB * B
EPR = E // R
EPR_PAD = NS * EPT
N_PAD = 10240
ROWS_PER_TILE = N_PAD // NS
ZCHUNKS = ROWS_PER_TILE // B


def _sc_aggregate_body(h_hbm, src_hbm, dst_hbm, zrow_hbm, out_hbm,
                       srcbuf, dstbuf, *bufs_and_sems):
    rows = bufs_and_sems[:NBUF]
    acc = bufs_and_sems[NBUF]
    gsems = bufs_and_sems[NBUF + 1:]
    c = lax.axis_index("c")
    s = lax.axis_index("s")
    for j in range(R // NC):
        r = c * (R // NC) + j
        pltpu.sync_copy(zrow_hbm, rows[0])
        zbase = s * ROWS_PER_TILE
        for kz in range(ZCHUNKS):
            pltpu.sync_copy(rows[0], acc.at[pl.ds(zbase + kz * B, B)])
        rem = ROWS_PER_TILE - ZCHUNKS * B
        if rem:
            pltpu.sync_copy(rows[0].at[pl.ds(0, rem)],
                            acc.at[pl.ds(zbase + ZCHUNKS * B, rem)])
        plsc.subcore_barrier()
        ebase = (r * NS + s) * EPT
        pltpu.sync_copy(src_hbm.at[pl.ds(ebase, EPT)], srcbuf)
        pltpu.sync_copy(dst_hbm.at[r, s], dstbuf)
        for q in range(NBUF):
            pltpu.async_copy(h_hbm.at[srcbuf.at[pl.ds(q * B, B)]],
                             rows[q], gsems[q])

        def quad(i, carry):
            for q in range(NBUF):
                k = NBUF * i + q
                pltpu.make_async_copy(h_hbm.at[srcbuf.at[pl.ds(k * B, B)]],
                                      rows[q], gsems[q]).wait()
                pltpu.sync_copy(rows[q], acc.at[dstbuf.at[k]], add=True)

                @pl.when(k + NBUF < NB)
                def _():
                    pltpu.async_copy(
                        h_hbm.at[srcbuf.at[pl.ds((k + NBUF) * B, B)]],
                        rows[q], gsems[q])

            return carry

        lax.fori_loop(0, NB // NBUF, quad, 0)
        plsc.subcore_barrier()
        pltpu.sync_copy(acc.at[pl.ds(s * ROWS_PER_TILE, ROWS_PER_TILE)],
                        out_hbm.at[r, pl.ds(s * ROWS_PER_TILE, ROWS_PER_TILE)])
        plsc.subcore_barrier()


_sc_aggregate = functools.partial(
    pl.kernel,
    out_type=jax.ShapeDtypeStruct((R, N_PAD, D), jnp.float32),
    mesh=plsc.VectorSubcoreMesh(core_axis_name="c", subcore_axis_name="s"),
    scratch_types=(
        [
            pltpu.VMEM((EPT,), jnp.int32),
            pltpu.VMEM((NB, B), jnp.int32),
        ]
        + [pltpu.VMEM((B, D), jnp.float32) for _ in range(NBUF)]
        + [pltpu.VMEM_SHARED((N_PAD, D), jnp.float32)]
        + [pltpu.SemaphoreType.DMA for _ in range(NBUF)]
    ),
)(_sc_aggregate_body)


BN = 1000


def _tc_layer_body(a_ref, h_ref, w_ref, l_ref, b_ref, o_ref):
    acc = jnp.dot(h_ref[...], l_ref[...], preferred_element_type=jnp.float32)
    for r in range(R):
        acc = acc + jnp.dot(a_ref[r], w_ref[r],
                            preferred_element_type=jnp.float32)
    o_ref[...] = jnp.maximum(acc + b_ref[...], 0.0)


def _tc_layer(A, h, W, loop, b):
    return pl.pallas_call(
        _tc_layer_body,
        grid=(N // BN,),
        in_specs=[
            pl.BlockSpec((R, BN, D), lambda i: (0, i, 0)),
            pl.BlockSpec((BN, D), lambda i: (i, 0)),
            pl.BlockSpec((R, D, H), lambda i: (0, 0, 0)),
            pl.BlockSpec((D, H), lambda i: (0, 0)),
            pl.BlockSpec((1, H), lambda i: (0, 0)),
        ],
        out_specs=pl.BlockSpec((BN, H), lambda i: (i, 0)),
        out_shape=jax.ShapeDtypeStruct((N, H), jnp.float32),
    )(A, h, W, loop, b)


def _tc_layer2_body(a_ref, h_ref, w_ref, l_ref, b_ref, wc_ref, bc_ref,
                    o_ref, m_ref):
    i = pl.program_id(0)
    acc = jnp.dot(h_ref[...], l_ref[...], preferred_element_type=jnp.float32)
    for r in range(R):
        acc = acc + jnp.dot(a_ref[r], w_ref[r],
                            preferred_element_type=jnp.float32)
    x = jnp.maximum(acc + b_ref[...], 0.0)
    bm = jnp.max(x, axis=0, keepdims=True)

    @pl.when(i == 0)
    def _():
        m_ref[...] = bm

    @pl.when(i > 0)
    def _():
        m_ref[...] = jnp.maximum(m_ref[...], bm)

    @pl.when(i == N // BN - 1)
    def _():
        o_ref[...] = jnp.dot(m_ref[...], wc_ref[...],
                             preferred_element_type=jnp.float32) + bc_ref[...]


def _tc_layer2(A, h, W, loop, b, Wc, bc):
    return pl.pallas_call(
        _tc_layer2_body,
        grid=(N // BN,),
        in_specs=[
            pl.BlockSpec((R, BN, D), lambda i: (0, i, 0)),
            pl.BlockSpec((BN, D), lambda i: (i, 0)),
            pl.BlockSpec((R, D, H), lambda i: (0, 0, 0)),
            pl.BlockSpec((D, H), lambda i: (0, 0)),
            pl.BlockSpec((1, H), lambda i: (0, 0)),
            pl.BlockSpec((H, C), lambda i: (0, 0)),
            pl.BlockSpec((1, C), lambda i: (0, 0)),
        ],
        out_specs=pl.BlockSpec((1, C), lambda i: (0, 0)),
        out_shape=jax.ShapeDtypeStruct((1, C), jnp.float32),
        scratch_shapes=[pltpu.VMEM((1, H), jnp.float32)],
    )(A, h, W, loop, b, Wc, bc)


def kernel(h, edge_index, W1, loop1, b1, W2, loop2, b2, Wc, bc):
    ei = edge_index.astype(jnp.int32)
    src4 = ei[0].reshape(EPR, R).T
    dst4 = ei[1].reshape(EPR, R).T
    src_p = jnp.pad(src4, ((0, 0), (0, EPR_PAD - EPR))).reshape(-1)
    dst_p = jnp.pad(dst4, ((0, 0), (0, EPR_PAD - EPR)),
                    constant_values=N).reshape(R, NS, NB, B)
    zrow = jnp.zeros((B, D), jnp.float32)
    b1r = b1.reshape(1, H)
    b2r = b2.reshape(1, H)
    bcr = bc.reshape(1, C)

    A1 = _sc_aggregate(h, src_p, dst_p, zrow)[:, :N, :]
    h1 = _tc_layer(A1, h, W1, loop1, b1r)
    A2 = _sc_aggregate(h1, src_p, dst_p, zrow)[:, :N, :]
    return _tc_layer2(A2, h1, W2, loop2, b2r, Wc, bcr)
